# Initial kernel scaffold; baseline (speedup 1.0000x reference)
import jax
import jax.numpy as jnp
from jax.experimental import pallas as pl


def kernel(logits, gumbel_u, mask, s, terminal):
    raise NotImplementedError("write your pallas kernel here")


# speedup vs baseline: 1.5780x; 1.5780x over previous
import functools

import jax
import jax.numpy as jnp
from jax.experimental import pallas as pl
from jax.experimental.pallas import tpu as pltpu

_VB = 32768
_NEG = float(jnp.finfo(jnp.float32).min)
_IMAX = 2**31 - 1


def _vocab_body(nblocks, vocab, mask_ref, x_ref, u_ref, act_ref, lp_ref,
                se_ref, bv_ref, bi_ref, cv_ref):
    j = pl.program_id(0)

    @pl.when(j == 0)
    def _init():
        se_ref[...] = jnp.zeros_like(se_ref)
        bv_ref[...] = jnp.full_like(bv_ref, -jnp.inf)
        bi_ref[...] = jnp.zeros_like(bi_ref)
        cv_ref[...] = jnp.full_like(cv_ref, _NEG)

    x = x_ref[...]
    u = u_ref[...]
    m = mask_ref[...] != 0
    col = jax.lax.broadcasted_iota(jnp.int32, x.shape, 1) + j * _VB
    bad = m | (col >= vocab)
    xm = jnp.where(bad, _NEG, x)
    se_ref[...] += jnp.sum(jnp.exp(xm), axis=1, keepdims=True)

    g = -jnp.log(-jnp.log(u + 1e-9) + 1e-9)
    t = jnp.where(bad, -jnp.inf, xm + g)
    bm = jnp.max(t, axis=1, keepdims=True)
    hit = (t == bm) & jnp.logical_not(bad)
    bidx = jnp.min(jnp.where(hit, col, _IMAX), axis=1, keepdims=True)
    cvs = jnp.max(jnp.where(col == bidx, xm, -jnp.inf), axis=1, keepdims=True)

    better = bm > bv_ref[...]
    bi_ref[...] = jnp.where(better, bidx, bi_ref[...])
    cv_ref[...] = jnp.where(better, cvs, cv_ref[...])
    bv_ref[...] = jnp.where(better, bm, bv_ref[...])

    @pl.when(j == nblocks - 1)
    def _fin():
        lse = jnp.log(se_ref[...])
        act_ref[...] = bi_ref[...]
        lp_ref[...] = cv_ref[...] - lse


def _reward_body(s_ref, t_ref, out_ref):
    s = s_ref[...]
    t = t_ref[...]
    e1 = jnp.sum(s * t, axis=1, keepdims=True)
    e2 = jnp.sum(jnp.abs(t - s) * s, axis=1, keepdims=True)
    er = jnp.exp(e1 - e2)
    er = jnp.where(jnp.isinf(er), 10000.0, er)
    r = (t - s) ** 2 + 1e-6
    mse = 1.0 / (jnp.sum(r, axis=1, keepdims=True) + 1.0)
    out_ref[...] = 0.7 * er + 0.3 * mse


def kernel(logits, gumbel_u, mask, s, terminal):
    b, vocab = logits.shape
    nblocks = pl.cdiv(vocab, _VB)
    mask2 = mask.astype(jnp.int32).reshape(1, vocab)

    acts, lp = pl.pallas_call(
        functools.partial(_vocab_body, nblocks, vocab),
        grid=(nblocks,),
        in_specs=[
            pl.BlockSpec((1, _VB), lambda j: (0, j)),
            pl.BlockSpec((b, _VB), lambda j: (0, j)),
            pl.BlockSpec((b, _VB), lambda j: (0, j)),
        ],
        out_specs=[
            pl.BlockSpec((b, 1), lambda j: (0, 0)),
            pl.BlockSpec((b, 1), lambda j: (0, 0)),
        ],
        out_shape=[
            jax.ShapeDtypeStruct((b, 1), jnp.int32),
            jax.ShapeDtypeStruct((b, 1), jnp.float32),
        ],
        scratch_shapes=[
            pltpu.VMEM((b, 1), jnp.float32),
            pltpu.VMEM((b, 1), jnp.float32),
            pltpu.VMEM((b, 1), jnp.int32),
            pltpu.VMEM((b, 1), jnp.float32),
        ],
        compiler_params=pltpu.CompilerParams(
            dimension_semantics=("arbitrary",)),
    )(mask2, logits, gumbel_u)

    hw = s.shape[1] * s.shape[2]
    ime = pl.pallas_call(
        _reward_body,
        out_shape=jax.ShapeDtypeStruct((b, 1), jnp.float32),
    )(s.reshape(b, hw), terminal.reshape(1, hw))

    return acts.reshape(b), lp.reshape(b), ime.reshape(b)
